# Initial kernel scaffold; baseline (speedup 1.0000x reference)
import jax
import jax.numpy as jnp
from jax.experimental import pallas as pl


def kernel(embedding, W0, b0, W1, b1, Wlr0, blr0, Wlr1, blr1, Wd0, bd0, gamma0, beta0, Wd1, bd1, w_pred, edge_index, edges):
    raise NotImplementedError("write your pallas kernel here")


# speedup vs baseline: 5.3715x; 5.3715x over previous
import functools

import jax
import jax.numpy as jnp
from jax import lax
from jax.experimental import pallas as pl
from jax.experimental.pallas import tpu as pltpu
from jax.experimental.pallas import tpu_sc as plsc

N = 10000
E = 160000
EQ = 8192
D = 256
HID = 256
K = 50
OUT = 12
KP = 128

RB = 1000
NBLK = N // RB

NP = 10240
DEG_T = NP // 16

ET = E // 16
EC = 80
NCH = ET // EC
RT = NP // 16

QT = EQ // 32
QC = 128

_f32 = jnp.float32
_i32 = jnp.int32
_HIGH = jax.lax.Precision.HIGHEST


def _mm(a, b):
    return lax.dot_general(a, b, (((1,), (0,)), ((), ())),
                           precision=_HIGH, preferred_element_type=_f32)


def _mmT(a, b):
    return lax.dot_general(a, b, (((0,), (0,)), ((), ())),
                           precision=_HIGH, preferred_element_type=_f32)


def _relu(x):
    return jnp.maximum(x, 0.0)


def _head_store(x, w_ref, wlr_ref, blr_ref, dinv, hsa_ref, hsb_ref, u_ref,
                t_ref, vtz_ref, vs_ref):
    h = _mm(x, w_ref[...])
    hs = h * dinv
    hsa_ref[...] = hs[:, :KP]
    hsb_ref[...] = hs[:, KP:]
    g = _relu(_mm(x, wlr_ref[...]) + blr_ref[...][None, :])
    u_ref[...] = g[:, :KP]
    v = g[:, KP:2 * KP]
    z = g[:, 2 * KP:3 * KP]
    t_ref[...] = g[:, 3 * KP:]
    vtz = _mmT(v, z)
    vs = jnp.broadcast_to(jnp.sum(v, axis=0, keepdims=True), (8, KP))

    @pl.when(pl.program_id(0) == 0)
    def _():
        vtz_ref[...] = jnp.zeros_like(vtz_ref)
        vs_ref[...] = jnp.zeros_like(vs_ref)

    vtz_ref[...] += vtz
    vs_ref[...] += vs


def _tck1_body(deg_ref, x_ref, w_ref, wlr_ref, blr_ref,
               hsa_ref, hsb_ref, u_ref, t_ref, vtz_ref, vs_ref):
    dinv = lax.rsqrt(deg_ref[...] + 1.0)
    _head_store(x_ref[...], w_ref, wlr_ref, blr_ref, dinv,
                hsa_ref, hsb_ref, u_ref, t_ref, vtz_ref, vs_ref)


def _tck3_body(deg_ref, y_ref, s1_ref, s2_ref, gamma_ref, beta_ref,
               w_ref, wlr_ref, blr_ref,
               hsa_ref, hsb_ref, u_ref, t_ref, vtz_ref, vs_ref):
    dinv = lax.rsqrt(deg_ref[...] + 1.0)
    mean = s1_ref[0:1, :] * (1.0 / N)
    var = s2_ref[0:1, :] * (1.0 / N) - mean * mean
    xn = (gamma_ref[...][None, :] * (y_ref[...] - mean)
          * lax.rsqrt(var + 1e-5) + beta_ref[...][None, :])
    _head_store(xn, w_ref, wlr_ref, blr_ref, dinv,
                hsa_ref, hsb_ref, u_ref, t_ref, vtz_ref, vs_ref)


def _tail_compute(deg_ref, sega_ref, segb_ref, hsa_ref, hsb_ref, u_ref, t_ref,
                  vtz_ref, vs_ref, wa_ref, wb_ref, wc_ref, bd_ref, bg_ref):
    dinv = lax.rsqrt(deg_ref[...] + 1.0)
    seg = jnp.concatenate([sega_ref[...], segb_ref[...]], axis=1)
    hs = jnp.concatenate([hsa_ref[...], hsb_ref[...]], axis=1)
    xl = _relu(dinv * (seg + hs) + bg_ref[...][None, :])
    u = u_ref[...]
    res = _mm(u, vtz_ref[...])
    denom = jnp.sum(u * vs_ref[0:1, :], axis=1, keepdims=True)
    rd = res / (denom + 1e-6)
    return (_mm(rd, wa_ref[...]) + _mm(t_ref[...], wb_ref[...])
            + _mm(xl, wc_ref[...]) + bd_ref[...][None, :])


def _tck2_body(deg_ref, sega_ref, segb_ref, hsa_ref, hsb_ref, u_ref, t_ref,
               vtz_ref, vs_ref, wa_ref, wb_ref, wc_ref, bd_ref, bg_ref,
               y_ref, s1_ref, s2_ref):
    y = _relu(_tail_compute(deg_ref, sega_ref, segb_ref, hsa_ref, hsb_ref,
                            u_ref, t_ref, vtz_ref, vs_ref, wa_ref, wb_ref,
                            wc_ref, bd_ref, bg_ref))
    y_ref[...] = y
    s1 = jnp.broadcast_to(jnp.sum(y, axis=0, keepdims=True), (8, HID))
    s2 = jnp.broadcast_to(jnp.sum(y * y, axis=0, keepdims=True), (8, HID))

    @pl.when(pl.program_id(0) == 0)
    def _():
        s1_ref[...] = jnp.zeros_like(s1_ref)
        s2_ref[...] = jnp.zeros_like(s2_ref)

    s1_ref[...] += s1
    s2_ref[...] += s2


def _tck4_body(deg_ref, sega_ref, segb_ref, hsa_ref, hsb_ref, u_ref, t_ref,
               vtz_ref, vs_ref, wa_ref, wb_ref, wc_ref, bd_ref, bg_ref,
               wp_ref, x2_ref, x2w_ref):
    x2 = _tail_compute(deg_ref, sega_ref, segb_ref, hsa_ref, hsb_ref,
                       u_ref, t_ref, vtz_ref, vs_ref, wa_ref, wb_ref,
                       wc_ref, bd_ref, bg_ref)
    x2_ref[...] = x2
    x2w_ref[...] = x2 * wp_ref[...][None, :]


def _row_spec(cols):
    return pl.BlockSpec((RB, cols), lambda i: (i, 0))


def _full_spec(shape):
    nd = len(shape)
    return pl.BlockSpec(shape, lambda i: (0,) * nd)


_SD = jax.ShapeDtypeStruct


def _tc_head_call(body, extra_in_specs, deg2d, *args):
    out_shapes = [_SD((N, KP), _f32), _SD((N, KP), _f32), _SD((N, KP), _f32),
                  _SD((N, KP), _f32), _SD((KP, KP), _f32), _SD((8, KP), _f32)]
    out_specs = [_row_spec(KP), _row_spec(KP), _row_spec(KP), _row_spec(KP),
                 _full_spec((KP, KP)), _full_spec((8, KP))]
    return pl.pallas_call(
        body,
        grid=(NBLK,),
        in_specs=[_row_spec(1)] + extra_in_specs,
        out_specs=out_specs,
        out_shape=out_shapes,
    )(deg2d, *args)


def _tc_layer0_head(deg2d, x, w0, wlrp0, blrp0):
    specs = [_row_spec(D), _full_spec((D, HID)), _full_spec((D, 4 * KP)),
             _full_spec((4 * KP,))]
    return _tc_head_call(_tck1_body, specs, deg2d, x, w0, wlrp0, blrp0)


def _tc_layer1_head(deg2d, y, s1, s2, gamma, beta, w1, wlrp1, blrp1):
    specs = [_row_spec(HID), _full_spec((8, HID)), _full_spec((8, HID)),
             _full_spec((HID,)), _full_spec((HID,)), _full_spec((HID, HID)),
             _full_spec((HID, 4 * KP)), _full_spec((4 * KP,))]
    return _tc_head_call(_tck3_body, specs, deg2d, y, s1, s2, gamma, beta,
                         w1, wlrp1, blrp1)


_TAIL_IN_SPECS = [_row_spec(1), _row_spec(KP), _row_spec(KP), _row_spec(KP),
                  _row_spec(KP), _row_spec(KP), _row_spec(KP),
                  _full_spec((KP, KP)), _full_spec((8, KP))]


def _tc_layer0_tail(deg2d, sega, segb, hsa, hsb, u, t, vtz, vs,
                    wa, wb, wc, bd, bg):
    specs = _TAIL_IN_SPECS + [_full_spec((KP, HID)), _full_spec((KP, HID)),
                              _full_spec((HID, HID)), _full_spec((HID,)),
                              _full_spec((HID,))]
    return pl.pallas_call(
        _tck2_body,
        grid=(NBLK,),
        in_specs=specs,
        out_specs=[_row_spec(HID), _full_spec((8, HID)), _full_spec((8, HID))],
        out_shape=[_SD((N, HID), _f32), _SD((8, HID), _f32),
                   _SD((8, HID), _f32)],
    )(deg2d, sega, segb, hsa, hsb, u, t, vtz, vs, wa, wb, wc, bd, bg)


def _tc_layer1_tail(deg2d, sega, segb, hsa, hsb, u, t, vtz, vs,
                    wa, wb, wc, bdp, bg, wp128):
    specs = _TAIL_IN_SPECS + [_full_spec((KP, KP)), _full_spec((KP, KP)),
                              _full_spec((HID, KP)), _full_spec((KP,)),
                              _full_spec((HID,)), _full_spec((KP,))]
    return pl.pallas_call(
        _tck4_body,
        grid=(NBLK,),
        in_specs=specs,
        out_specs=[_row_spec(KP), _row_spec(KP)],
        out_shape=[_SD((N, KP), _f32), _SD((N, KP), _f32)],
    )(deg2d, sega, segb, hsa, hsb, u, t, vtz, vs, wa, wb, wc, bdp, bg, wp128)


@functools.cache
def _sc_mesh():
    return plsc.VectorSubcoreMesh(core_axis_name="c", subcore_axis_name="s")


@functools.cache
def _sc_degree_kernel():
    return pl.kernel(
        _sc_degree_body,
        out_type=_SD((NP,), _f32),
        mesh=_sc_mesh(),
        compiler_params=pltpu.CompilerParams(needs_layout_passes=False),
        scratch_types=[
            pltpu.VMEM((EC,), _i32),
            pltpu.VMEM((EC,), _f32),
            pltpu.VMEM((DEG_T,), _f32),
            pltpu.VMEM_SHARED((NP,), _f32),
            pltpu.SemaphoreType.DMA,
        ],
    )


def _sc_degree_body(dst_hbm, zeros_hbm, ones_hbm, deg_out,
                    dst_v, ones_v, row_v, acc_sh, sem):
    c = lax.axis_index("c")
    s = lax.axis_index("s")

    @pl.when(c == 0)
    def _():
        base_r = s * DEG_T
        pltpu.sync_copy(zeros_hbm.at[pl.ds(base_r, DEG_T)],
                        acc_sh.at[pl.ds(base_r, DEG_T)])
        pltpu.sync_copy(ones_hbm, ones_v)
        plsc.subcore_barrier()

        def chunk(k, carry):
            base = s * ET + k * EC
            pltpu.sync_copy(dst_hbm.at[pl.ds(base, EC)], dst_v)
            pltpu.sync_copy(ones_v, acc_sh.at[dst_v], add=True)
            return carry

        lax.fori_loop(0, NCH, chunk, 0)
        plsc.subcore_barrier()
        pltpu.sync_copy(acc_sh.at[pl.ds(base_r, DEG_T)], row_v)
        pltpu.sync_copy(row_v, deg_out.at[pl.ds(base_r, DEG_T)])


@functools.cache
def _sc_segsum_kernel():
    return pl.kernel(
        _sc_segsum_body,
        out_type=[_SD((NP, KP), _f32), _SD((NP, KP), _f32)],
        mesh=_sc_mesh(),
        compiler_params=pltpu.CompilerParams(needs_layout_passes=False),
        scratch_types=[
            pltpu.VMEM((EC,), _i32),
            pltpu.VMEM((EC,), _i32),
            pltpu.VMEM((EC, KP), _f32),
            pltpu.VMEM_SHARED((NP, KP), _f32),
            pltpu.SemaphoreType.DMA,
        ],
    )


def _sc_segsum_body(hsa_hbm, hsb_hbm, src_hbm, dst_hbm, zeros_hbm, sega, segb,
                    src_v, dst_v, rows_v, acc_sh, sem):
    c = lax.axis_index("c")
    s = lax.axis_index("s")
    base_r = s * RT
    pltpu.sync_copy(zeros_hbm.at[pl.ds(base_r, RT)],
                    acc_sh.at[pl.ds(base_r, RT)])
    plsc.subcore_barrier()

    def chunk_from(table):
        def chunk(k, carry):
            base = s * ET + k * EC
            pltpu.sync_copy(src_hbm.at[pl.ds(base, EC)], src_v)
            pltpu.sync_copy(dst_hbm.at[pl.ds(base, EC)], dst_v)
            pltpu.async_copy(table.at[src_v], rows_v, sem).wait()
            pltpu.sync_copy(rows_v, acc_sh.at[dst_v], add=True)
            return carry
        lax.fori_loop(0, NCH, chunk, 0)

    @pl.when(c == 0)
    def _():
        chunk_from(hsa_hbm)

    @pl.when(c == 1)
    def _():
        chunk_from(hsb_hbm)

    plsc.subcore_barrier()

    @pl.when(c == 0)
    def _():
        pltpu.sync_copy(acc_sh.at[pl.ds(base_r, RT)],
                        sega.at[pl.ds(base_r, RT)])

    @pl.when(c == 1)
    def _():
        pltpu.sync_copy(acc_sh.at[pl.ds(base_r, RT)],
                        segb.at[pl.ds(base_r, RT)])


@functools.cache
def _sc_score_kernel():
    return pl.kernel(
        _sc_score_body,
        out_type=_SD((EQ,), _f32),
        mesh=_sc_mesh(),
        compiler_params=pltpu.CompilerParams(needs_layout_passes=False),
        scratch_types=[
            pltpu.VMEM((QC,), _i32),
            pltpu.VMEM((QC,), _i32),
            pltpu.VMEM((QC, KP), _f32),
            pltpu.VMEM((QC, KP), _f32),
            pltpu.VMEM((QT,), _f32),
            pltpu.SemaphoreType.DMA,
        ],
    )


def _sc_score_body(x2w_hbm, x2_hbm, qs_hbm, qd_hbm, scores,
                   qs_v, qd_v, rs_v, rd_v, out_v, sem):
    c = lax.axis_index("c")
    s = lax.axis_index("s")
    w = s * 2 + c

    lanes = lax.iota(_i32, 16)

    def chunk(ch, carry):
        base = w * QT + ch * QC
        pltpu.sync_copy(qs_hbm.at[pl.ds(base, QC)], qs_v)
        pltpu.sync_copy(qd_hbm.at[pl.ds(base, QC)], qd_v)
        pltpu.async_copy(x2w_hbm.at[qs_v], rs_v, sem).wait()
        pltpu.async_copy(x2_hbm.at[qd_v], rd_v, sem).wait()

        def group(g, carry2):
            e_ids = g * 16 + lanes
            acc = jnp.zeros((16,), _f32)
            for f in range(16):
                f_ids = jnp.full((16,), f, _i32)
                a = plsc.load_gather(rs_v, [e_ids, f_ids])
                b = plsc.load_gather(rd_v, [e_ids, f_ids])
                acc = acc + a * b
            out_v[pl.ds(ch * QC + g * 16, 16)] = acc
            return carry2

        lax.fori_loop(0, QC // 16, group, 0)
        return carry

    lax.fori_loop(0, QT // QC, chunk, 0)
    pltpu.sync_copy(out_v, scores.at[pl.ds(w * QT, QT)])


def _pad_lr(wlr, blr):
    din = wlr.shape[0]
    wp = jnp.zeros((din, 4 * KP), _f32)
    bp = jnp.zeros((4 * KP,), _f32)
    for i in range(4):
        wp = wp.at[:, i * KP:i * KP + K].set(wlr[:, i * K:(i + 1) * K])
        bp = bp.at[i * KP:i * KP + K].set(blr[i * K:(i + 1) * K])
    return wp, bp


def _pad_wd(wd, dout_pad):
    dout = wd.shape[1]
    wa = jnp.zeros((KP, dout_pad), _f32).at[:K, :dout].set(wd[:K])
    wb = jnp.zeros((KP, dout_pad), _f32).at[:K, :dout].set(wd[K:2 * K])
    wc = jnp.zeros((wd.shape[0] - 2 * K, dout_pad), _f32)
    wc = wc.at[:, :dout].set(wd[2 * K:])
    return wa, wb, wc


def kernel(embedding, W0, b0, W1, b1, Wlr0, blr0, Wlr1, blr1, Wd0, bd0,
           gamma0, beta0, Wd1, bd1, w_pred, edge_index, edges):
    src = edge_index[0].astype(_i32)
    dst = edge_index[1].astype(_i32)
    qs = edges[0].astype(_i32)
    qd = edges[1].astype(_i32)

    wlrp0, blrp0 = _pad_lr(Wlr0, blr0)
    wlrp1, blrp1 = _pad_lr(Wlr1, blr1)
    wa0, wb0, wc0 = _pad_wd(Wd0, HID)
    wa1, wb1, wc1 = _pad_wd(Wd1, KP)
    bd1p = jnp.zeros((KP,), _f32).at[:OUT].set(bd1)
    wp128 = jnp.zeros((KP,), _f32).at[:OUT].set(w_pred)

    zeros_np = jnp.zeros((NP,), _f32)
    ones_ec = jnp.ones((EC,), _f32)
    zeros_nkp = jnp.zeros((NP, KP), _f32)

    deg = _sc_degree_kernel()(dst, zeros_np, ones_ec)
    deg2d = deg[:N].reshape(N, 1)

    hsa0, hsb0, u0, t0, vtz0, vs0 = _tc_layer0_head(
        deg2d, embedding, W0, wlrp0, blrp0)
    sega0, segb0 = _sc_segsum_kernel()(hsa0, hsb0, src, dst, zeros_nkp)
    y, s1, s2 = _tc_layer0_tail(deg2d, sega0, segb0, hsa0, hsb0, u0, t0,
                                vtz0, vs0, wa0, wb0, wc0, bd0, b0)

    hsa1, hsb1, u1, t1, vtz1, vs1 = _tc_layer1_head(
        deg2d, y, s1, s2, gamma0, beta0, W1, wlrp1, blrp1)
    sega1, segb1 = _sc_segsum_kernel()(hsa1, hsb1, src, dst, zeros_nkp)
    x2, x2w = _tc_layer1_tail(deg2d, sega1, segb1, hsa1, hsb1, u1, t1,
                              vtz1, vs1, wa1, wb1, wc1, bd1p, b1, wp128)

    return _sc_score_kernel()(x2w, x2, qs, qd)
